# baseline (speedup 1.0000x reference)
import functools

import jax
import jax.numpy as jnp
from jax import lax
from jax.experimental import pallas as pl
from jax.experimental.pallas import tpu as pltpu
from jax.experimental.pallas import tpu_sc as plsc

_NC = 2
_NS = 16
_NW = _NC * _NS
_LANES = 16


@functools.cache
def _build(B, N, D, V):
    R = B * N
    per_w = R // _NW
    CHUNK = 2 * N
    n_chunks = per_w // CHUNK
    sub = []
    o = 0
    while o < CHUNK:
        s = min(128, CHUNK - o)
        sub.append((o, s))
        o += s
    mesh = plsc.VectorSubcoreMesh(core_axis_name="c", subcore_axis_name="s")

    idx_scratch = [pltpu.VMEM((s,), jnp.int32) for (_, s) in sub]

    @functools.partial(
        pl.kernel,
        out_type=jax.ShapeDtypeStruct((R, D), jnp.float32),
        mesh=mesh,
        compiler_params=pltpu.CompilerParams(use_tc_tiling_on_sc=False),
        scratch_types=[
            *idx_scratch,
            pltpu.VMEM((CHUNK, D), jnp.float32),
            pltpu.VMEM((N, D), jnp.float32),
            pltpu.SemaphoreType.DMA,
        ],
    )
    def emb(x_hbm, table_hbm, pos_hbm, out_hbm, *rest):
        idx_refs = rest[: len(sub)]
        rows_v, pos_v, sem = rest[len(sub):]
        wid = lax.axis_index("s") * _NC + lax.axis_index("c")
        base = wid * per_w
        pltpu.sync_copy(pos_hbm, pos_v)

        def chunk_body(g, carry):
            off = base + g * CHUNK
            for (so, sl), iref in zip(sub, idx_refs):
                pltpu.sync_copy(x_hbm.at[pl.ds(off + so, sl)], iref)
            cps = [
                pltpu.async_copy(table_hbm.at[iref], rows_v.at[pl.ds(so, sl)], sem)
                for (so, sl), iref in zip(sub, idx_refs)
            ]
            for cp in cps:
                cp.wait()

            def add_body(r, c2):
                for c in range(D // _LANES):
                    dsl = pl.ds(c * _LANES, _LANES)
                    p = pos_v[r, dsl]
                    rows_v[r, dsl] += p
                    rows_v[r + N, dsl] += p
                return c2

            lax.fori_loop(0, N, add_body, 0)
            pltpu.sync_copy(rows_v, out_hbm.at[pl.ds(off, CHUNK)])
            return carry

        lax.fori_loop(0, n_chunks, chunk_body, 0)

    return emb


def kernel(x, table, pos_embedding):
    B, N = x.shape
    V, D = table.shape
    xf = x.reshape(-1).astype(jnp.int32)
    out = _build(B, N, D, V)(xf, table, pos_embedding)
    return out.reshape(B, N, D)
